# baseline (speedup 1.0000x reference)
import functools

import jax
import jax.numpy as jnp
from jax.experimental import pallas as pl
from jax.experimental.pallas import tpu as pltpu

_EPS_BN = 1e-5
_DP = jax.lax.Precision.DEFAULT


def _body(pf_ref, sf1_ref, w1a_ref, b1a_ref, w1b_ref, b1b_ref,
          gamma_ref, beta_ref, sxyz_ref, pxt_ref, sf2_ref,
          out_ref, tmp1_ref, acc_ref, bn_ref, feat2_ref,
          *, nb, nt, n1, n2, tile):
    g = pl.program_id(0)

    @pl.when(g == 0)
    def _init():
        acc_ref[...] = jnp.zeros_like(acc_ref)

    @pl.when(g < nb)
    def _phase1():
        t1 = jnp.dot(pf_ref[0], w1a_ref[...], precision=_DP,
                     preferred_element_type=jnp.float32) + b1a_ref[...]
        tmp1_ref[pl.ds(g * n2, n2), :] = t1
        acc_ref[0:1, :] += jnp.sum(t1, axis=0, keepdims=True)
        acc_ref[1:2, :] += jnp.sum(t1 * t1, axis=0, keepdims=True)
        t2 = jnp.dot(sf1_ref[0], w1b_ref[...], precision=_DP,
                     preferred_element_type=jnp.float32) + b1b_ref[...]
        acc_ref[2:3, :] += jnp.sum(t2, axis=0, keepdims=True)
        acc_ref[3:4, :] += jnp.sum(t2 * t2, axis=0, keepdims=True)

        @pl.when(g == nb - 1)
        def _finalize():
            gamma = gamma_ref[...]
            beta = beta_ref[...]
            mean1 = acc_ref[0:1, :] / float(nb * n2)
            var1 = acc_ref[1:2, :] / float(nb * n2) - mean1 * mean1
            s1 = gamma * jax.lax.rsqrt(var1 + _EPS_BN)
            bn_ref[0:1, :] = s1
            bn_ref[1:2, :] = beta - mean1 * s1
            mean2 = acc_ref[2:3, :] / float(nb * n1)
            var2 = acc_ref[3:4, :] / float(nb * n1) - mean2 * mean2
            s2 = gamma * jax.lax.rsqrt(var2 + _EPS_BN)
            bn_ref[2:3, :] = s2
            bn_ref[3:4, :] = beta - mean2 * s2

    @pl.when(g >= nb)
    def _phase2():
        gg = g - nb
        b = gg // nt

        @pl.when(gg % nt == 0)
        def _norm_feat2():
            feat2_ref[...] = jnp.maximum(
                tmp1_ref[pl.ds(b * n2, n2), :] * bn_ref[0:1, :]
                + bn_ref[1:2, :], 0.0)

        sx = sxyz_ref[0]
        px = pxt_ref[0]
        a2 = jnp.sum(sx * sx, axis=1, keepdims=True)
        b2 = jnp.sum(px * px, axis=0, keepdims=True)
        ab = jnp.dot(sx, px, precision=_DP,
                     preferred_element_type=jnp.float32)
        sq = jnp.maximum(a2 + b2 - 2.0 * ab, 1e-12)

        iota = jax.lax.broadcasted_iota(
            jnp.int32, sq.shape, 1).astype(jnp.float32)
        m1 = jnp.min(sq, axis=1, keepdims=True)
        i1 = jnp.min(jnp.where(sq == m1, iota, jnp.inf),
                     axis=1, keepdims=True)
        sq1 = jnp.where(iota == i1, jnp.inf, sq)
        m2 = jnp.min(sq1, axis=1, keepdims=True)
        i2 = jnp.min(jnp.where(sq1 == m2, iota, jnp.inf),
                     axis=1, keepdims=True)
        sq2 = jnp.where(iota == i2, jnp.inf, sq1)
        m3 = jnp.min(sq2, axis=1, keepdims=True)
        i3 = jnp.min(jnp.where(sq2 == m3, iota, jnp.inf),
                     axis=1, keepdims=True)
        p1 = jnp.sqrt(m1) + 1e-8
        p2 = jnp.sqrt(m2) + 1e-8
        p3 = jnp.sqrt(m3) + 1e-8
        q12 = p1 * p2
        q23 = p2 * p3
        q13 = p1 * p3
        denom = 1.0 / (q23 + q13 + q12)
        wn1 = q23 * denom
        wn2 = q13 * denom
        wn3 = q12 * denom
        s_mat = jnp.where(iota == i1, wn1,
                          jnp.where(iota == i2, wn2,
                                    jnp.where(iota == i3, wn3, 0.0)))

        interp = jnp.dot(s_mat, feat2_ref[...], precision=_DP,
                         preferred_element_type=jnp.float32)
        t2 = jnp.dot(sf2_ref[0], w1b_ref[...], precision=_DP,
                     preferred_element_type=jnp.float32) + b1b_ref[...]
        out_ref[0] = interp + jnp.maximum(
            t2 * bn_ref[2:3, :] + bn_ref[3:4, :], 0.0)


@jax.jit
def kernel(points_xyz, points_features, skipped_xyz, skipped_features,
           W1a, b1a, W1b, b1b, gamma, beta):
    B, N2, Cin = points_features.shape
    _, N1, C = skipped_features.shape
    TILE = 1024
    NT = N1 // TILE
    G = B + B * NT

    b1a2 = b1a.reshape(1, C)
    b1b2 = b1b.reshape(1, C)
    gamma2 = gamma.reshape(1, C)
    beta2 = beta.reshape(1, C)
    pxt = jnp.transpose(points_xyz, (0, 2, 1))

    def p1_map(g):
        return (jnp.minimum(g, B - 1), 0, 0)

    def p2b(g):
        return jnp.maximum(g - B, 0) // NT

    def p2t(g):
        return jnp.maximum(g - B, 0) % NT

    vec = pl.BlockSpec((1, C), lambda g: (0, 0))
    body = functools.partial(_body, nb=B, nt=NT, n1=N1, n2=N2, tile=TILE)
    out = pl.pallas_call(
        body,
        grid=(G,),
        in_specs=[
            pl.BlockSpec((1, N2, Cin), p1_map),
            pl.BlockSpec((1, N1, C), p1_map),
            pl.BlockSpec((Cin, C), lambda g: (0, 0)),
            vec, pl.BlockSpec((C, C), lambda g: (0, 0)), vec, vec, vec,
            pl.BlockSpec((1, TILE, 3), lambda g: (p2b(g), p2t(g), 0)),
            pl.BlockSpec((1, 3, N2), lambda g: (p2b(g), 0, 0)),
            pl.BlockSpec((1, TILE, C), lambda g: (p2b(g), p2t(g), 0)),
        ],
        out_specs=pl.BlockSpec((1, TILE, C), lambda g: (p2b(g), p2t(g), 0)),
        out_shape=jax.ShapeDtypeStruct((B, N1, C), jnp.float32),
        scratch_shapes=[
            pltpu.VMEM((B * N2, C), jnp.float32),
            pltpu.VMEM((4, C), jnp.float32),
            pltpu.VMEM((4, C), jnp.float32),
            pltpu.VMEM((N2, C), jnp.float32),
        ],
    )(points_features, skipped_features, W1a, b1a2, W1b, b1b2, gamma2, beta2,
      skipped_xyz, pxt, skipped_features)

    return (skipped_xyz, out)


# speedup vs baseline: 1.0085x; 1.0085x over previous
import functools

import jax
import jax.numpy as jnp
from jax.experimental import pallas as pl
from jax.experimental.pallas import tpu as pltpu

_EPS_BN = 1e-5
_DP = jax.lax.Precision.DEFAULT


def _body(pf_ref, sf1_ref, w1a_ref, b1a_ref, w1b_ref, b1b_ref,
          gamma_ref, beta_ref, sxyz_ref, pxt_ref, sf2_ref,
          out_ref, tmp1_ref, acc_ref, bn_ref, feat2_ref,
          *, nb, nt, n1, n2, tile):
    g = pl.program_id(0)

    @pl.when(g == 0)
    def _init():
        acc_ref[...] = jnp.zeros_like(acc_ref)

    @pl.when(g < nb)
    def _phase1():
        t1 = jnp.dot(pf_ref[0], w1a_ref[...], precision=_DP,
                     preferred_element_type=jnp.float32) + b1a_ref[...]
        tmp1_ref[pl.ds(g * n2, n2), :] = t1
        acc_ref[0:1, :] += jnp.sum(t1, axis=0, keepdims=True)
        acc_ref[1:2, :] += jnp.sum(t1 * t1, axis=0, keepdims=True)
        t2 = jnp.dot(sf1_ref[0], w1b_ref[...], precision=_DP,
                     preferred_element_type=jnp.float32) + b1b_ref[...]
        acc_ref[2:3, :] += jnp.sum(t2, axis=0, keepdims=True)
        acc_ref[3:4, :] += jnp.sum(t2 * t2, axis=0, keepdims=True)

        @pl.when(g == nb - 1)
        def _finalize():
            gamma = gamma_ref[...]
            beta = beta_ref[...]
            mean1 = acc_ref[0:1, :] / float(nb * n2)
            var1 = acc_ref[1:2, :] / float(nb * n2) - mean1 * mean1
            s1 = gamma * jax.lax.rsqrt(var1 + _EPS_BN)
            bn_ref[0:1, :] = s1
            bn_ref[1:2, :] = beta - mean1 * s1
            mean2 = acc_ref[2:3, :] / float(nb * n1)
            var2 = acc_ref[3:4, :] / float(nb * n1) - mean2 * mean2
            s2 = gamma * jax.lax.rsqrt(var2 + _EPS_BN)
            bn_ref[2:3, :] = s2
            bn_ref[3:4, :] = beta - mean2 * s2

    @pl.when(g >= nb)
    def _phase2():
        gg = g - nb
        b = gg // nt

        @pl.when(gg % nt == 0)
        def _norm_feat2():
            feat2_ref[...] = jnp.maximum(
                tmp1_ref[pl.ds(b * n2, n2), :] * bn_ref[0:1, :]
                + bn_ref[1:2, :], 0.0)

        sx = sxyz_ref[0]
        px = pxt_ref[0]
        a2 = jnp.sum(sx * sx, axis=1, keepdims=True)
        b2 = jnp.sum(px * px, axis=0, keepdims=True)
        ab = jnp.dot(sx, px, precision=_DP,
                     preferred_element_type=jnp.float32)
        sq = jnp.maximum(a2 + b2 - 2.0 * ab, 1e-12)

        iota = jax.lax.broadcasted_iota(
            jnp.int32, sq.shape, 1).astype(jnp.float32)
        m1 = jnp.min(sq, axis=1, keepdims=True)
        i1 = jnp.min(jnp.where(sq == m1, iota, jnp.inf),
                     axis=1, keepdims=True)
        sq1 = jnp.where(iota == i1, jnp.inf, sq)
        m2 = jnp.min(sq1, axis=1, keepdims=True)
        i2 = jnp.min(jnp.where(sq1 == m2, iota, jnp.inf),
                     axis=1, keepdims=True)
        sq2 = jnp.where(iota == i2, jnp.inf, sq1)
        m3 = jnp.min(sq2, axis=1, keepdims=True)
        i3 = jnp.min(jnp.where(sq2 == m3, iota, jnp.inf),
                     axis=1, keepdims=True)
        p1 = jnp.sqrt(m1) + 1e-8
        p2 = jnp.sqrt(m2) + 1e-8
        p3 = jnp.sqrt(m3) + 1e-8
        q12 = p1 * p2
        q23 = p2 * p3
        q13 = p1 * p3
        denom = 1.0 / (q23 + q13 + q12)
        wn1 = q23 * denom
        wn2 = q13 * denom
        wn3 = q12 * denom
        s_mat = jnp.where(iota == i1, wn1,
                          jnp.where(iota == i2, wn2,
                                    jnp.where(iota == i3, wn3, 0.0)))

        interp = jnp.dot(s_mat, feat2_ref[...], precision=_DP,
                         preferred_element_type=jnp.float32)
        t2 = jnp.dot(sf2_ref[0], w1b_ref[...], precision=_DP,
                     preferred_element_type=jnp.float32) + b1b_ref[...]
        out_ref[0] = interp + jnp.maximum(
            t2 * bn_ref[2:3, :] + bn_ref[3:4, :], 0.0)


@jax.jit
def kernel(points_xyz, points_features, skipped_xyz, skipped_features,
           W1a, b1a, W1b, b1b, gamma, beta):
    B, N2, Cin = points_features.shape
    _, N1, C = skipped_features.shape
    TILE = 2048
    NT = N1 // TILE
    G = B + B * NT

    b1a2 = b1a.reshape(1, C)
    b1b2 = b1b.reshape(1, C)
    gamma2 = gamma.reshape(1, C)
    beta2 = beta.reshape(1, C)
    pxt = jnp.transpose(points_xyz, (0, 2, 1))

    def p1_map(g):
        return (jnp.minimum(g, B - 1), 0, 0)

    def p2b(g):
        return jnp.maximum(g - B, 0) // NT

    def p2t(g):
        return jnp.maximum(g - B, 0) % NT

    vec = pl.BlockSpec((1, C), lambda g: (0, 0))
    body = functools.partial(_body, nb=B, nt=NT, n1=N1, n2=N2, tile=TILE)
    out = pl.pallas_call(
        body,
        grid=(G,),
        in_specs=[
            pl.BlockSpec((1, N2, Cin), p1_map),
            pl.BlockSpec((1, N1, C), p1_map),
            pl.BlockSpec((Cin, C), lambda g: (0, 0)),
            vec, pl.BlockSpec((C, C), lambda g: (0, 0)), vec, vec, vec,
            pl.BlockSpec((1, TILE, 3), lambda g: (p2b(g), p2t(g), 0)),
            pl.BlockSpec((1, 3, N2), lambda g: (p2b(g), 0, 0)),
            pl.BlockSpec((1, TILE, C), lambda g: (p2b(g), p2t(g), 0)),
        ],
        out_specs=pl.BlockSpec((1, TILE, C), lambda g: (p2b(g), p2t(g), 0)),
        out_shape=jax.ShapeDtypeStruct((B, N1, C), jnp.float32),
        scratch_shapes=[
            pltpu.VMEM((B * N2, C), jnp.float32),
            pltpu.VMEM((4, C), jnp.float32),
            pltpu.VMEM((4, C), jnp.float32),
            pltpu.VMEM((N2, C), jnp.float32),
        ],
    )(points_features, skipped_features, W1a, b1a2, W1b, b1b2, gamma2, beta2,
      skipped_xyz, pxt, skipped_features)

    return (skipped_xyz, out)
